# baseline (speedup 1.0000x reference)
import functools

import jax
import jax.numpy as jnp
from jax.experimental import pallas as pl
from jax.experimental.pallas import tpu as pltpu

VOCAB = 8192
DIM = 256
BM = 512


def _vote_kernel(h_ref, cbt_ref, out_ref):
    h = h_ref[...]
    s = jnp.where(h < 0, -1.0, 1.0).astype(jnp.bfloat16)
    out_ref[...] = jnp.dot(s, cbt_ref[...], preferred_element_type=jnp.float32)


@jax.jit
def kernel(h, codebook):
    b, t, d = h.shape
    m = b * t
    h2 = h.reshape(m, d)
    cbt = codebook.T.astype(jnp.bfloat16)
    grid = (m // BM,)
    out = pl.pallas_call(
        _vote_kernel,
        grid=grid,
        in_specs=[
            pl.BlockSpec((BM, d), lambda i: (i, 0)),
            pl.BlockSpec((d, VOCAB), lambda i: (0, 0)),
        ],
        out_specs=pl.BlockSpec((BM, VOCAB), lambda i: (i, 0)),
        out_shape=jax.ShapeDtypeStruct((m, VOCAB), jnp.float32),
        compiler_params=pltpu.CompilerParams(
            dimension_semantics=("parallel",),
        ),
    )(h2, cbt)
    return out.reshape(b, t, VOCAB)


# speedup vs baseline: 1.0730x; 1.0730x over previous
import functools

import jax
import jax.numpy as jnp
from jax.experimental import pallas as pl
from jax.experimental.pallas import tpu as pltpu

VOCAB = 8192
DIM = 256
BM = 512


def _vote_kernel(h_ref, cb_ref, out_ref):
    h = h_ref[...]
    s = jnp.where(h < 0, -1.0, 1.0).astype(jnp.bfloat16)
    cb = cb_ref[...].astype(jnp.bfloat16)
    out_ref[...] = jax.lax.dot_general(
        s, cb, (((1,), (1,)), ((), ())), preferred_element_type=jnp.float32)


@jax.jit
def kernel(h, codebook):
    b, t, d = h.shape
    m = b * t
    h2 = h.reshape(m, d)
    grid = (m // BM,)
    out = pl.pallas_call(
        _vote_kernel,
        grid=grid,
        in_specs=[
            pl.BlockSpec((BM, d), lambda i: (i, 0)),
            pl.BlockSpec((VOCAB, d), lambda i: (0, 0)),
        ],
        out_specs=pl.BlockSpec((BM, VOCAB), lambda i: (i, 0)),
        out_shape=jax.ShapeDtypeStruct((m, VOCAB), jnp.float32),
        compiler_params=pltpu.CompilerParams(
            dimension_semantics=("parallel",),
        ),
    )(h2, codebook)
    return out.reshape(b, t, VOCAB)


# speedup vs baseline: 1.0772x; 1.0039x over previous
import functools

import jax
import jax.numpy as jnp
from jax.experimental import pallas as pl
from jax.experimental.pallas import tpu as pltpu

VOCAB = 8192
DIM = 256
BM = 256


def _vote_kernel(h_ref, cb_ref, out_ref):
    h = h_ref[...]
    s = jnp.where(h < 0, -1.0, 1.0).astype(jnp.bfloat16)
    cb = cb_ref[...].astype(jnp.bfloat16)
    out_ref[...] = jax.lax.dot_general(
        s, cb, (((1,), (1,)), ((), ())), preferred_element_type=jnp.float32)


@jax.jit
def kernel(h, codebook):
    b, t, d = h.shape
    m = b * t
    h2 = h.reshape(m, d)
    grid = (m // BM,)
    out = pl.pallas_call(
        _vote_kernel,
        grid=grid,
        in_specs=[
            pl.BlockSpec((BM, d), lambda i: (i, 0)),
            pl.BlockSpec((VOCAB, d), lambda i: (0, 0)),
        ],
        out_specs=pl.BlockSpec((BM, VOCAB), lambda i: (i, 0)),
        out_shape=jax.ShapeDtypeStruct((m, VOCAB), jnp.float32),
        compiler_params=pltpu.CompilerParams(
            dimension_semantics=("parallel",),
        ),
    )(h2, codebook)
    return out.reshape(b, t, VOCAB)
